# baseline (speedup 1.0000x reference)
import functools

import jax
import jax.numpy as jnp
from jax import lax
from jax.experimental import pallas as pl
from jax.experimental.pallas import tpu as pltpu
from jax.experimental.pallas import tpu_sc as plsc

N = 100000
IN = 128
HID = 64
K = 3
G = 1024
BN = 4000
SP = 16
NEG = -1e30


def _leaky(v):
    return jnp.where(v > 0, v, 0.01 * v)


def _k1_body(x_ref, b_ref, w1_ref, b1_ref, w2_ref, b2_ref, s_out, m_out):
    step = pl.program_id(0)

    @pl.when(step == 0)
    def _():
        m_out[...] = jnp.full((8, G), NEG, jnp.float32)

    x = x_ref[...]
    h = _leaky(jnp.dot(x, w1_ref[...], preferred_element_type=jnp.float32)
               + b1_ref[...])
    s = jnp.dot(h, w2_ref[...], preferred_element_type=jnp.float32) + b2_ref[...]
    bf = b_ref[...].astype(jnp.float32)
    lane = lax.broadcasted_iota(jnp.int32, (1, SP), 1)
    s_out[...] = jnp.where(lane == K, bf, s)

    seg = lax.broadcasted_iota(jnp.int32, (1, G), 1).astype(jnp.float32)
    oh = bf == seg
    reds = []
    for k in range(K):
        vals = jnp.where(oh, s[:, k:k + 1], NEG)
        reds.append(jnp.max(vals, axis=0, keepdims=True))
    upd = jnp.concatenate(reds + [jnp.full((8 - K, G), NEG, jnp.float32)], axis=0)
    m_out[...] = jnp.maximum(m_out[...], upd)


def _k1(x, batch2d, W1, b1, W2p, b2p):
    grid = N // BN
    return pl.pallas_call(
        _k1_body,
        grid=(grid,),
        in_specs=[
            pl.BlockSpec((BN, IN), lambda i: (i, 0)),
            pl.BlockSpec((BN, 1), lambda i: (i, 0)),
            pl.BlockSpec((IN, HID), lambda i: (0, 0)),
            pl.BlockSpec((1, HID), lambda i: (0, 0)),
            pl.BlockSpec((HID, SP), lambda i: (0, 0)),
            pl.BlockSpec((1, SP), lambda i: (0, 0)),
        ],
        out_specs=[
            pl.BlockSpec((BN, SP), lambda i: (i, 0)),
            pl.BlockSpec((8, G), lambda i: (0, 0)),
        ],
        out_shape=[
            jax.ShapeDtypeStruct((N, SP), jnp.float32),
            jax.ShapeDtypeStruct((8, G), jnp.float32),
        ],
    )(x, batch2d, W1, b1, W2p, b2p)


def _k2_body(s_ref, m_ref, mt_ref, d_out, i_out):
    step = pl.program_id(0)

    @pl.when(step == 0)
    def _():
        d_out[...] = jnp.zeros((G, 8), jnp.float32)
        i_out[...] = jnp.full((8, G), -1, jnp.int32)

    sc = s_ref[...]
    bf = sc[:, K:K + 1]
    seg = lax.broadcasted_iota(jnp.int32, (1, G), 1).astype(jnp.float32)
    oh = bf == seg
    rowid = step * BN + lax.broadcasted_iota(jnp.int32, (BN, 1), 0)

    ohf = oh.astype(jnp.float32)
    es, cs, ired = [], [], []
    for k in range(K):
        sk = sc[:, k:k + 1]
        c = jnp.max(sk)
        es.append(jnp.exp(sk - c))
        cs.append(jnp.broadcast_to(c.reshape(1, 1), (1, 1)))
        cand = jnp.where(oh & (sk == m_ref[k:k + 1, :]), rowid, -1)
        ired.append(jnp.max(cand, axis=0, keepdims=True))
    e8 = jnp.concatenate(es + [jnp.zeros((BN, 8 - K), jnp.float32)], axis=1)
    c8 = jnp.concatenate(cs + [jnp.zeros((1, 8 - K), jnp.float32)], axis=1)
    dupd = lax.dot_general(ohf, e8, (((0,), (0,)), ((), ())),
                           preferred_element_type=jnp.float32)
    mt = mt_ref[...]
    scale = jnp.where(mt > 0.5 * NEG, jnp.exp(c8 - mt), 0.0)
    iupd = jnp.concatenate(ired + [jnp.full((8 - K, G), -1, jnp.int32)], axis=0)
    d_out[...] = d_out[...] + dupd * scale
    i_out[...] = jnp.maximum(i_out[...], iupd)


def _k2(scores16, M, MT):
    grid = N // BN
    return pl.pallas_call(
        _k2_body,
        grid=(grid,),
        in_specs=[
            pl.BlockSpec((BN, SP), lambda i: (i, 0)),
            pl.BlockSpec((8, G), lambda i: (0, 0)),
            pl.BlockSpec((G, 8), lambda i: (0, 0)),
        ],
        out_specs=[
            pl.BlockSpec((G, 8), lambda i: (0, 0)),
            pl.BlockSpec((8, G), lambda i: (0, 0)),
        ],
        out_shape=[
            jax.ShapeDtypeStruct((G, 8), jnp.float32),
            jax.ShapeDtypeStruct((8, G), jnp.int32),
        ],
    )(scores16, M, MT)


def _sc_gather(x, idx):
    info = plsc.get_sparse_core_info()
    nw = info.num_cores * info.num_subcores
    b = K * G
    bpw = b // nw
    mesh = plsc.VectorSubcoreMesh(core_axis_name="c", subcore_axis_name="s")

    @functools.partial(
        pl.kernel, mesh=mesh,
        out_type=jax.ShapeDtypeStruct((b, IN), jnp.float32),
        scratch_types=[
            pltpu.VMEM((bpw,), jnp.int32),
            pltpu.VMEM((bpw, IN), jnp.float32),
            pltpu.SemaphoreType.DMA,
        ],
    )
    def gk(x_hbm, idx_hbm, gx_hbm, idx_v, rows_v, sem):
        wid = lax.axis_index("s") * info.num_cores + lax.axis_index("c")
        base = wid * bpw
        pltpu.sync_copy(idx_hbm.at[pl.ds(base, bpw)], idx_v)
        pltpu.async_copy(x_hbm.at[idx_v], rows_v, sem).wait()
        pltpu.sync_copy(rows_v, gx_hbm.at[pl.ds(base, bpw)])

    return gk(x, idx)


def _k3_body(gx_ref, idxt_ref, srow0_ref, mt_ref, dt_ref, w3_ref, b3_ref,
             o_ref):
    segcol = lax.broadcasted_iota(jnp.int32, (G, 1), 0).astype(jnp.float32)
    b0 = srow0_ref[0, K]
    acc = jnp.zeros((G, IN), jnp.float32)
    for k in range(K):
        empty = idxt_ref[:, k:k + 1] < 0
        wne = 1.0 / (dt_ref[:, k:k + 1] + 1e-16)
        m0 = jnp.sum(jnp.where(segcol == b0, mt_ref[:, k:k + 1], 0.0))
        d0 = jnp.sum(jnp.where(segcol == b0, dt_ref[:, k:k + 1], 0.0))
        w0 = jnp.exp(srow0_ref[0, k] - m0) / (d0 + 1e-16)
        w = jnp.where(empty, w0, wne)
        xk = gx_ref[k * G:(k + 1) * G, :] * w
        acc = acc + jnp.dot(xk, w3_ref[k * IN:(k + 1) * IN, :],
                            preferred_element_type=jnp.float32)
    o_ref[...] = _leaky(acc + b3_ref[...])


def _k3(gx, IDXT, srow0, MT, DT, W3, b3):
    return pl.pallas_call(
        _k3_body,
        out_shape=jax.ShapeDtypeStruct((G, IN), jnp.float32),
    )(gx, IDXT, srow0, MT, DT, W3, b3.reshape(1, IN))


def kernel(x, batch, num_graphs, W1, b1, W2, b2, W3, b3):
    W2p = jnp.zeros((HID, SP), jnp.float32).at[:, :K].set(W2)
    b2p = jnp.zeros((1, SP), jnp.float32).at[0, :K].set(b2)
    batch2d = batch.astype(jnp.int32).reshape(N, 1)

    scores16, M = _k1(x, batch2d, W1, b1.reshape(1, HID), W2p, b2p)
    MT = M.T
    D, IDX = _k2(scores16, M, MT)
    idx = jnp.clip(IDX[:K].reshape(K * G), 0, N - 1).astype(jnp.int32)
    gx = _sc_gather(x, idx)
    out = _k3(gx, IDX.T, scores16[0:1, :], MT, D, W3, b3)
    return out + (jnp.asarray(num_graphs) * 0).astype(out.dtype)


# speedup vs baseline: 1.0104x; 1.0104x over previous
import functools

import jax
import jax.numpy as jnp
from jax import lax
from jax.experimental import pallas as pl
from jax.experimental.pallas import tpu as pltpu
from jax.experimental.pallas import tpu_sc as plsc

N = 100000
IN = 128
HID = 64
K = 3
G = 1024
BN = 2000
SP = 16
NEG = -1e30


def _leaky(v):
    return jnp.where(v > 0, v, 0.01 * v)


def _k1_body(x_ref, b_ref, w1_ref, b1_ref, w2_ref, b2_ref, s_out, m_out):
    step = pl.program_id(0)

    @pl.when(step == 0)
    def _():
        m_out[...] = jnp.full((8, G), NEG, jnp.float32)

    x = x_ref[...]
    h = _leaky(jnp.dot(x, w1_ref[...], preferred_element_type=jnp.float32)
               + b1_ref[...])
    s = jnp.dot(h, w2_ref[...], preferred_element_type=jnp.float32) + b2_ref[...]
    bf = b_ref[...].astype(jnp.float32)
    lane = lax.broadcasted_iota(jnp.int32, (1, SP), 1)
    s_out[...] = jnp.where(lane == K, bf, s)

    seg = lax.broadcasted_iota(jnp.int32, (1, G), 1).astype(jnp.float32)
    oh = bf == seg
    reds = []
    for k in range(K):
        vals = jnp.where(oh, s[:, k:k + 1], NEG)
        reds.append(jnp.max(vals, axis=0, keepdims=True))
    upd = jnp.concatenate(reds + [jnp.full((8 - K, G), NEG, jnp.float32)], axis=0)
    m_out[...] = jnp.maximum(m_out[...], upd)


def _k1(x, batch2d, W1, b1, W2p, b2p):
    grid = N // BN
    return pl.pallas_call(
        _k1_body,
        grid=(grid,),
        in_specs=[
            pl.BlockSpec((BN, IN), lambda i: (i, 0)),
            pl.BlockSpec((BN, 1), lambda i: (i, 0)),
            pl.BlockSpec((IN, HID), lambda i: (0, 0)),
            pl.BlockSpec((1, HID), lambda i: (0, 0)),
            pl.BlockSpec((HID, SP), lambda i: (0, 0)),
            pl.BlockSpec((1, SP), lambda i: (0, 0)),
        ],
        out_specs=[
            pl.BlockSpec((BN, SP), lambda i: (i, 0)),
            pl.BlockSpec((8, G), lambda i: (0, 0)),
        ],
        out_shape=[
            jax.ShapeDtypeStruct((N, SP), jnp.float32),
            jax.ShapeDtypeStruct((8, G), jnp.float32),
        ],
    )(x, batch2d, W1, b1, W2p, b2p)


def _k2_body(s_ref, m_ref, mt_ref, d_out, i_out):
    step = pl.program_id(0)

    @pl.when(step == 0)
    def _():
        d_out[...] = jnp.zeros((G, 8), jnp.float32)
        i_out[...] = jnp.full((8, G), -1, jnp.int32)

    sc = s_ref[...]
    bf = sc[:, K:K + 1]
    seg = lax.broadcasted_iota(jnp.int32, (1, G), 1).astype(jnp.float32)
    oh = bf == seg
    rowid = step * BN + lax.broadcasted_iota(jnp.int32, (BN, 1), 0)

    ohf = oh.astype(jnp.float32)
    es, cs, ired = [], [], []
    for k in range(K):
        sk = sc[:, k:k + 1]
        c = jnp.max(sk)
        es.append(jnp.exp(sk - c))
        cs.append(jnp.broadcast_to(c.reshape(1, 1), (1, 1)))
        cand = jnp.where(oh & (sk == m_ref[k:k + 1, :]), rowid, -1)
        ired.append(jnp.max(cand, axis=0, keepdims=True))
    e8 = jnp.concatenate(es + [jnp.zeros((BN, 8 - K), jnp.float32)], axis=1)
    c8 = jnp.concatenate(cs + [jnp.zeros((1, 8 - K), jnp.float32)], axis=1)
    dupd = lax.dot_general(ohf, e8, (((0,), (0,)), ((), ())),
                           preferred_element_type=jnp.float32)
    mt = mt_ref[...]
    scale = jnp.where(mt > 0.5 * NEG, jnp.exp(c8 - mt), 0.0)
    iupd = jnp.concatenate(ired + [jnp.full((8 - K, G), -1, jnp.int32)], axis=0)
    d_out[...] = d_out[...] + dupd * scale
    i_out[...] = jnp.maximum(i_out[...], iupd)


def _k2(scores16, M, MT):
    grid = N // BN
    return pl.pallas_call(
        _k2_body,
        grid=(grid,),
        in_specs=[
            pl.BlockSpec((BN, SP), lambda i: (i, 0)),
            pl.BlockSpec((8, G), lambda i: (0, 0)),
            pl.BlockSpec((G, 8), lambda i: (0, 0)),
        ],
        out_specs=[
            pl.BlockSpec((G, 8), lambda i: (0, 0)),
            pl.BlockSpec((8, G), lambda i: (0, 0)),
        ],
        out_shape=[
            jax.ShapeDtypeStruct((G, 8), jnp.float32),
            jax.ShapeDtypeStruct((8, G), jnp.int32),
        ],
    )(scores16, M, MT)


def _sc_gather(x, idx):
    info = plsc.get_sparse_core_info()
    nw = info.num_cores * info.num_subcores
    b = K * G
    bpw = b // nw
    mesh = plsc.VectorSubcoreMesh(core_axis_name="c", subcore_axis_name="s")

    @functools.partial(
        pl.kernel, mesh=mesh,
        out_type=jax.ShapeDtypeStruct((b, IN), jnp.float32),
        scratch_types=[
            pltpu.VMEM((bpw,), jnp.int32),
            pltpu.VMEM((bpw, IN), jnp.float32),
            pltpu.SemaphoreType.DMA,
        ],
    )
    def gk(x_hbm, idx_hbm, gx_hbm, idx_v, rows_v, sem):
        wid = lax.axis_index("s") * info.num_cores + lax.axis_index("c")
        base = wid * bpw
        pltpu.sync_copy(idx_hbm.at[pl.ds(base, bpw)], idx_v)
        pltpu.async_copy(x_hbm.at[idx_v], rows_v, sem).wait()
        pltpu.sync_copy(rows_v, gx_hbm.at[pl.ds(base, bpw)])

    return gk(x, idx)


def _k3_body(gx_ref, idxt_ref, srow0_ref, mt_ref, dt_ref, w3_ref, b3_ref,
             o_ref):
    segcol = lax.broadcasted_iota(jnp.int32, (G, 1), 0).astype(jnp.float32)
    b0 = srow0_ref[0, K]
    acc = jnp.zeros((G, IN), jnp.float32)
    for k in range(K):
        empty = idxt_ref[:, k:k + 1] < 0
        wne = 1.0 / (dt_ref[:, k:k + 1] + 1e-16)
        m0 = jnp.sum(jnp.where(segcol == b0, mt_ref[:, k:k + 1], 0.0))
        d0 = jnp.sum(jnp.where(segcol == b0, dt_ref[:, k:k + 1], 0.0))
        w0 = jnp.exp(srow0_ref[0, k] - m0) / (d0 + 1e-16)
        w = jnp.where(empty, w0, wne)
        xk = gx_ref[k * G:(k + 1) * G, :] * w
        acc = acc + jnp.dot(xk, w3_ref[k * IN:(k + 1) * IN, :],
                            preferred_element_type=jnp.float32)
    o_ref[...] = _leaky(acc + b3_ref[...])


def _k3(gx, IDXT, srow0, MT, DT, W3, b3):
    return pl.pallas_call(
        _k3_body,
        out_shape=jax.ShapeDtypeStruct((G, IN), jnp.float32),
    )(gx, IDXT, srow0, MT, DT, W3, b3.reshape(1, IN))


def kernel(x, batch, num_graphs, W1, b1, W2, b2, W3, b3):
    W2p = jnp.zeros((HID, SP), jnp.float32).at[:, :K].set(W2)
    b2p = jnp.zeros((1, SP), jnp.float32).at[0, :K].set(b2)
    batch2d = batch.astype(jnp.int32).reshape(N, 1)

    scores16, M = _k1(x, batch2d, W1, b1.reshape(1, HID), W2p, b2p)
    MT = M.T
    D, IDX = _k2(scores16, M, MT)
    idx = jnp.clip(IDX[:K].reshape(K * G), 0, N - 1).astype(jnp.int32)
    gx = _sc_gather(x, idx)
    out = _k3(gx, IDX.T, scores16[0:1, :], MT, D, W3, b3)
    return out + (jnp.asarray(num_graphs) * 0).astype(out.dtype)
